# baseline (speedup 1.0000x reference)
import jax
import jax.numpy as jnp
from jax.experimental import pallas as pl

K = 1024
D = 10
N = 8192
BLK = 2048
GRID = N // BLK


def _vq_kernel(x_ref, w_ref, out_ref, loss_ref):
    pid = pl.program_id(0)
    x = x_ref[...]
    wt = w_ref[...].T
    acc = jnp.zeros((BLK, K), dtype=jnp.float32)
    for d in range(D):
        diff = x[:, d][:, None] - wt[d, :][None, :]
        acc = acc + diff * diff
    idx = jnp.argmin(acc, axis=1)
    onehot = (jax.lax.broadcasted_iota(jnp.int32, (BLK, K), 1)
              == idx[:, None]).astype(jnp.float32)
    q = jax.lax.dot_general(
        onehot, w_ref[...],
        dimension_numbers=(((1,), (0,)), ((), ())),
        preferred_element_type=jnp.float32)
    out_ref[...] = x + (q - x)
    partial = jnp.sum((x - q) ** 2).reshape(1, 1)

    @pl.when(pid == 0)
    def _():
        loss_ref[...] = jnp.zeros((1, 1), jnp.float32)

    loss_ref[...] += partial

    @pl.when(pid == GRID - 1)
    def _():
        loss_ref[...] = loss_ref[...] / (N * D)


@jax.jit
def kernel(encoder_embedding, embedding_weight):
    out, loss = pl.pallas_call(
        _vq_kernel,
        grid=(GRID,),
        in_specs=[
            pl.BlockSpec((BLK, D), lambda i: (i, 0)),
            pl.BlockSpec((K, D), lambda i: (0, 0)),
        ],
        out_specs=[
            pl.BlockSpec((BLK, D), lambda i: (i, 0)),
            pl.BlockSpec((1, 1), lambda i: (0, 0)),
        ],
        out_shape=[
            jax.ShapeDtypeStruct((N, D), jnp.float32),
            jax.ShapeDtypeStruct((1, 1), jnp.float32),
        ],
    )(encoder_embedding, embedding_weight)
    return out, loss[0, 0]
